# baseline (speedup 1.0000x reference)
import functools

import jax
import jax.numpy as jnp
from jax import lax
from jax.experimental import pallas as pl
from jax.experimental.pallas import tpu as pltpu
from jax.experimental.pallas import tpu_sc as plsc

_L = 16
_CAP = 8192
_HB = 64
_NR = 8
_SEG = 4096


@functools.cache
def _make_lookup(V, D, B):
    info = plsc.get_sparse_core_info()
    NC, NS = info.num_cores, info.num_subcores
    NW = NC * NS
    n_tc = (V + 127) // 128
    tc_per_w = (n_tc + NW - 1) // NW
    n_vec = B // _L
    nc_pad = ((tc_per_w + 2 + _L) // _L) * _L
    mesh = plsc.VectorSubcoreMesh(core_axis_name="c", subcore_axis_name="s")

    @functools.partial(
        pl.kernel,
        mesh=mesh,
        out_type=jax.ShapeDtypeStruct((B, 128), jnp.float32),
        compiler_params=pltpu.CompilerParams(needs_layout_passes=False),
        scratch_types=[
            pltpu.VMEM((2, _SEG), jnp.int32),
            pltpu.VMEM((_CAP + _L,), jnp.int32),
            pltpu.VMEM((_CAP + _L,), jnp.int32),
            pltpu.VMEM((_CAP + _L,), jnp.int32),
            pltpu.VMEM((_CAP + _L,), jnp.int32),
            pltpu.VMEM((nc_pad,), jnp.int32),
            pltpu.SMEM((tc_per_w + 2,), jnp.int32),
            pltpu.VMEM((_NR // 2, D, 256), jnp.float32),
            pltpu.VMEM((2, _HB, 128), jnp.float32),
            pltpu.VMEM((2, _HB), jnp.int32),
            pltpu.SemaphoreType.DMA,
            pltpu.SemaphoreType.DMA,
            pltpu.SemaphoreType.DMA,
        ],
    )
    def k(tableT_hbm, idx_hbm, out_hbm, seg_v, hv, hp, uv, up, cnt_v, cur,
          col_v, rows_v, rpos_v, isem, csem, osem):
        wid = lax.axis_index("s") * NC + lax.axis_index("c")
        tc0 = wid * tc_per_w
        lo = tc0 * 128
        hi = jnp.minimum((tc0 + tc_per_w) * 128, V)
        iota = lax.iota(jnp.int32, _L)
        lane0 = iota == 0

        d4 = [iota + k16 * _L for k16 in range(4)]
        cmax = (n_tc - 2) * 128

        def fetch_col(pr):
            base = jnp.minimum(lo + pr * 256, cmax)
            base = pl.multiple_of(base, 128)
            return pltpu.async_copy(
                tableT_hbm.at[:, pl.ds(base, 256)],
                col_v.at[jnp.bitwise_and(pr, _NR // 2 - 1)], csem)

        def wait_col():
            pltpu.make_async_copy(
                tableT_hbm.at[:, pl.ds(0, 256)], col_v.at[0], csem).wait()

        def flush(fs):
            return pltpu.async_copy(
                rows_v.at[fs], out_hbm.at[rpos_v.at[fs]], osem)

        def wait_flush():
            pltpu.make_async_copy(
                out_hbm.at[pl.ds(0, _HB)], rows_v.at[0], osem).wait()

        def fetch_seg(s):
            return pltpu.async_copy(
                idx_hbm.at[pl.ds(s * _SEG, _SEG)],
                seg_v.at[jnp.bitwise_and(s, 1)], isem)

        def wait_seg():
            pltpu.make_async_copy(
                idx_hbm.at[pl.ds(0, _SEG)], seg_v.at[0], isem).wait()

        for c in range(_NR // 2 - 1):
            fetch_col(jnp.int32(c))
        fetch_seg(jnp.int32(0))

        def segbody(sg, cnt):
            @pl.when(sg + 1 < B // _SEG)
            def _():
                fetch_seg(sg + 1)

            wait_seg()
            slot = jnp.bitwise_and(sg, 1)

            def fbody(i, cnt):
                v16 = seg_v[slot, pl.ds(i * _L, _L)]
                m = (v16 >= lo) & (v16 < hi)
                p16 = iota + (sg * _SEG + i * _L)
                base = jnp.minimum(cnt, _CAP - _L)
                mi = m.astype(jnp.int32)
                s_inc = plsc.cumsum(mi)
                rank = s_inc - mi
                plsc.store_scatter(uv, [base + rank], v16, mask=m)
                plsc.store_scatter(up, [base + rank], p16, mask=m)
                return cnt + s_inc[_L - 1]

            return lax.fori_loop(0, _SEG // _L, fbody, cnt, unroll=4)

        nhit = lax.fori_loop(0, B // _SEG, segbody, jnp.int32(0))
        nhit = jnp.minimum(nhit, _CAP)
        n_hv = (nhit + _L - 1) // _L

        def hzero(g, _):
            cnt_v[pl.ds(g * _L, _L)] = jnp.zeros((_L,), jnp.int32)
            return 0
        lax.fori_loop(0, nc_pad // _L, hzero, 0)

        ones = jnp.ones((_L,), jnp.int32)

        def hbody(g, _):
            v16 = uv[pl.ds(g * _L, _L)]
            c16 = (v16 - lo) >> 7
            m = iota < (nhit - g * _L)
            c16 = jnp.where(m, c16, tc_per_w)
            plsc.addupdate_scatter(cnt_v, [c16], ones)
            return 0
        lax.fori_loop(0, n_hv, hbody, 0)

        def psbody(g, carry):
            c16 = cnt_v[pl.ds(g * _L, _L)]
            s16 = plsc.cumsum(c16) + carry
            cnt_v[pl.ds(g * _L, _L)] = s16
            return s16[_L - 1]
        lax.fori_loop(0, nc_pad // _L, psbody, jnp.int32(0))

        def smcopy(g, _):
            s16 = cnt_v[pl.ds(g * _L, _L)]
            for j in range(_L):
                if_ = pl.when(g * _L + j < tc_per_w + 2)

                @if_
                def _():
                    cur[g * _L + j] = s16[j]
            return 0
        lax.fori_loop(0, nc_pad // _L, smcopy, 0)

        def sstore(ref, i, val):
            plsc.store_scatter(ref, [jnp.full((_L,), i, jnp.int32)],
                               jnp.full((_L,), val, jnp.int32), mask=lane0)

        def sbody(g, _):
            p16 = up[pl.ds(g * _L, _L)]
            v16 = uv[pl.ds(g * _L, _L)]
            c16 = (v16 - lo) >> 7
            m = iota < (nhit - g * _L)
            c16 = jnp.where(m, c16, tc_per_w)
            s16 = jnp.zeros((_L,), jnp.int32)
            for j in range(_L):
                c = c16[j]
                s = cur[c] - 1
                cur[c] = s
                s16 = jnp.where(iota == j, s, s16)
            plsc.store_scatter(hv, [s16], v16)
            plsc.store_scatter(hp, [s16], p16)
            return 0
        lax.fori_loop(0, n_hv, sbody, 0)

        n_pair = (tc_per_w + 1) // 2

        def colbody(pr, carry):
            nb, fs, out, h0 = carry

            @pl.when(pr + _NR // 2 - 1 < n_pair)
            def _():
                fetch_col(pr + _NR // 2 - 1)

            wait_col()
            h1 = cur[jnp.minimum(2 * pr + 2, tc_per_w)]
            pbase = pr * 256

            def hitbody(h, carry):
                nb, fs, out, slot = carry
                v = hv[pl.ds(h, _L)][0]
                p = hp[pl.ds(h, _L)][0]
                l = (v - lo) - pbase
                for k16 in range(4):
                    row16 = plsc.load_gather(
                        col_v.at[slot],
                        [d4[k16], jnp.full((_L,), l, jnp.int32)])
                    rows_v[fs, nb, pl.ds(k16 * _L, _L)] = row16
                sstore(rpos_v.at[fs], nb, p)

                def do_flush(args):
                    nb, fs, out = args
                    flush(fs)
                    out = out + 1

                    def drain(out):
                        wait_flush()
                        return out - 1

                    out = lax.cond(out == 2, drain, lambda o: o, out)
                    return jnp.int32(0), 1 - fs, out

                nb, fs, out = lax.cond(
                    nb == _HB - 1, do_flush,
                    lambda a: (a[0] + 1, a[1], a[2]), (nb, fs, out))
                return nb, fs, out, slot

            nb, fs, out, _ = lax.fori_loop(
                h0, h1, hitbody,
                (nb, fs, out, jnp.bitwise_and(pr, _NR // 2 - 1)))
            return nb, fs, out, h1

        nb, fs, out, _ = lax.fori_loop(
            0, n_pair, colbody,
            (jnp.int32(0), jnp.int32(0), jnp.int32(0), jnp.int32(0)))

        def tbody(j, _):
            @pl.when(j >= nb)
            def _():
                sstore(rpos_v.at[fs], j, rpos_v[fs, pl.ds(0, _L)][0])
                for k16 in range(4):
                    rows_v[fs, j, pl.ds(k16 * _L, _L)] = (
                        rows_v[fs, 0, pl.ds(k16 * _L, _L)])
            return 0

        def tail_do(out):
            lax.fori_loop(0, _HB, tbody, 0)
            flush(fs)
            return out + 1

        out = lax.cond(nb > 0, tail_do, lambda o: o, out)

        def dbody(i, _):
            wait_flush()
            return 0
        lax.fori_loop(0, out, dbody, 0)

    return k


def kernel(x, table):
    B0, H, W, C = x.shape
    V, D = table.shape
    flat = x.astype(jnp.int32).reshape(-1)
    B = flat.shape[0]
    tableT = table.T
    out128 = _make_lookup(V, D, B)(tableT, flat)
    return out128[:, :D].reshape(B0, H, W, D)


# speedup vs baseline: 1.0103x; 1.0103x over previous
import functools

import jax
import jax.numpy as jnp
from jax import lax
from jax.experimental import pallas as pl
from jax.experimental.pallas import tpu as pltpu
from jax.experimental.pallas import tpu_sc as plsc

_L = 16
_CAP = 8192
_HB = 64
_NR = 8
_SEG = 4096


@functools.cache
def _make_lookup(V, D, B):
    info = plsc.get_sparse_core_info()
    NC, NS = info.num_cores, info.num_subcores
    NW = NC * NS
    n_tc = (V + 127) // 128
    tc_per_w = (n_tc + NW - 1) // NW
    n_vec = B // _L
    nc_pad = ((tc_per_w + 2 + _L) // _L) * _L
    mesh = plsc.VectorSubcoreMesh(core_axis_name="c", subcore_axis_name="s")

    @functools.partial(
        pl.kernel,
        mesh=mesh,
        out_type=jax.ShapeDtypeStruct((B, 128), jnp.float32),
        compiler_params=pltpu.CompilerParams(needs_layout_passes=False),
        scratch_types=[
            pltpu.VMEM((2, _SEG), jnp.int32),
            pltpu.VMEM((_CAP + _L,), jnp.int32),
            pltpu.VMEM((_CAP + _L,), jnp.int32),
            pltpu.VMEM((_CAP + _L,), jnp.int32),
            pltpu.VMEM((_CAP + _L,), jnp.int32),
            pltpu.VMEM((nc_pad,), jnp.int32),
            pltpu.SMEM((tc_per_w + 2,), jnp.int32),
            pltpu.VMEM((_NR, D, 128), jnp.float32),
            pltpu.VMEM((2, _HB, 128), jnp.float32),
            pltpu.VMEM((2, _HB), jnp.int32),
            pltpu.SemaphoreType.DMA,
            pltpu.SemaphoreType.DMA,
            pltpu.SemaphoreType.DMA,
        ],
    )
    def k(tableT_hbm, idx_hbm, out_hbm, seg_v, hv, hp, uv, up, cnt_v, cur,
          col_v, rows_v, rpos_v, isem, csem, osem):
        wid = lax.axis_index("s") * NC + lax.axis_index("c")
        tc0 = wid * tc_per_w
        lo = tc0 * 128
        hi = jnp.minimum((tc0 + tc_per_w) * 128, V)
        iota = lax.iota(jnp.int32, _L)
        lane0 = iota == 0

        d4 = [iota + k16 * _L for k16 in range(4)]
        cmax = (n_tc - 1) * 128

        def fetch_col(c):
            base = jnp.minimum((tc0 + c) * 128, cmax)
            base = pl.multiple_of(base, 128)
            return pltpu.async_copy(
                tableT_hbm.at[:, pl.ds(base, 128)],
                col_v.at[jnp.bitwise_and(c, _NR - 1)], csem)

        def wait_col():
            pltpu.make_async_copy(
                tableT_hbm.at[:, pl.ds(0, 128)], col_v.at[0], csem).wait()

        def flush(fs):
            return pltpu.async_copy(
                rows_v.at[fs], out_hbm.at[rpos_v.at[fs]], osem)

        def wait_flush():
            pltpu.make_async_copy(
                out_hbm.at[pl.ds(0, _HB)], rows_v.at[0], osem).wait()

        def fetch_seg(s):
            return pltpu.async_copy(
                idx_hbm.at[pl.ds(s * _SEG, _SEG)],
                seg_v.at[jnp.bitwise_and(s, 1)], isem)

        def wait_seg():
            pltpu.make_async_copy(
                idx_hbm.at[pl.ds(0, _SEG)], seg_v.at[0], isem).wait()

        for c in range(_NR - 1):
            fetch_col(jnp.int32(c))
        fetch_seg(jnp.int32(0))

        def segbody(sg, cnt):
            @pl.when(sg + 1 < B // _SEG)
            def _():
                fetch_seg(sg + 1)

            wait_seg()
            slot = jnp.bitwise_and(sg, 1)

            def fbody(i, cnt):
                v16 = seg_v[slot, pl.ds(i * _L, _L)]
                m = (v16 >= lo) & (v16 < hi)
                p16 = iota + (sg * _SEG + i * _L)
                base = jnp.minimum(cnt, _CAP - _L)
                mi = m.astype(jnp.int32)
                s_inc = plsc.cumsum(mi)
                rank = s_inc - mi
                plsc.store_scatter(uv, [base + rank], v16, mask=m)
                plsc.store_scatter(up, [base + rank], p16, mask=m)
                return cnt + s_inc[_L - 1]

            return lax.fori_loop(0, _SEG // _L, fbody, cnt, unroll=8)

        nhit = lax.fori_loop(0, B // _SEG, segbody, jnp.int32(0))
        nhit = jnp.minimum(nhit, _CAP)
        n_hv = (nhit + _L - 1) // _L

        def hzero(g, _):
            cnt_v[pl.ds(g * _L, _L)] = jnp.zeros((_L,), jnp.int32)
            return 0
        lax.fori_loop(0, nc_pad // _L, hzero, 0)

        ones = jnp.ones((_L,), jnp.int32)

        def hbody(g, _):
            v16 = uv[pl.ds(g * _L, _L)]
            c16 = (v16 - lo) >> 7
            m = iota < (nhit - g * _L)
            c16 = jnp.where(m, c16, tc_per_w)
            plsc.addupdate_scatter(cnt_v, [c16], ones)
            return 0
        lax.fori_loop(0, n_hv, hbody, 0)

        def psbody(g, carry):
            c16 = cnt_v[pl.ds(g * _L, _L)]
            s16 = plsc.cumsum(c16) + carry
            cnt_v[pl.ds(g * _L, _L)] = s16
            return s16[_L - 1]
        lax.fori_loop(0, nc_pad // _L, psbody, jnp.int32(0))

        def smcopy(g, _):
            s16 = cnt_v[pl.ds(g * _L, _L)]
            for j in range(_L):
                if_ = pl.when(g * _L + j < tc_per_w + 2)

                @if_
                def _():
                    cur[g * _L + j] = s16[j]
            return 0
        lax.fori_loop(0, nc_pad // _L, smcopy, 0)

        def sstore(ref, i, val):
            plsc.store_scatter(ref, [jnp.full((_L,), i, jnp.int32)],
                               jnp.full((_L,), val, jnp.int32), mask=lane0)

        def sbody(g, _):
            p16 = up[pl.ds(g * _L, _L)]
            v16 = uv[pl.ds(g * _L, _L)]
            c16 = (v16 - lo) >> 7
            m = iota < (nhit - g * _L)
            c16 = jnp.where(m, c16, tc_per_w)
            s16 = jnp.zeros((_L,), jnp.int32)
            for j in range(_L):
                c = c16[j]
                s = cur[c] - 1
                cur[c] = s
                s16 = jnp.where(iota == j, s, s16)
            plsc.store_scatter(hv, [s16], v16)
            plsc.store_scatter(hp, [s16], p16)
            return 0
        lax.fori_loop(0, n_hv, sbody, 0)

        def colbody(c, carry):
            nb, fs, out, h0 = carry

            @pl.when(c + _NR - 1 < tc_per_w)
            def _():
                fetch_col(c + _NR - 1)

            wait_col()
            h1 = cur[c + 1]

            def hitbody(h, carry):
                nb, fs, out, slot = carry
                v = hv[pl.ds(h, _L)][0]
                p = hp[pl.ds(h, _L)][0]
                l = v & 127
                for k16 in range(4):
                    row16 = plsc.load_gather(
                        col_v.at[slot],
                        [d4[k16], jnp.full((_L,), l, jnp.int32)])
                    rows_v[fs, nb, pl.ds(k16 * _L, _L)] = row16
                sstore(rpos_v.at[fs], nb, p)

                def do_flush(args):
                    nb, fs, out = args
                    flush(fs)
                    out = out + 1

                    def drain(out):
                        wait_flush()
                        return out - 1

                    out = lax.cond(out == 2, drain, lambda o: o, out)
                    return jnp.int32(0), 1 - fs, out

                nb, fs, out = lax.cond(
                    nb == _HB - 1, do_flush,
                    lambda a: (a[0] + 1, a[1], a[2]), (nb, fs, out))
                return nb, fs, out, slot

            nb, fs, out, _ = lax.fori_loop(
                h0, h1, hitbody, (nb, fs, out, jnp.bitwise_and(c, _NR - 1)))
            return nb, fs, out, h1

        nb, fs, out, _ = lax.fori_loop(
            0, tc_per_w, colbody,
            (jnp.int32(0), jnp.int32(0), jnp.int32(0), jnp.int32(0)))

        def tbody(j, _):
            @pl.when(j >= nb)
            def _():
                sstore(rpos_v.at[fs], j, rpos_v[fs, pl.ds(0, _L)][0])
                for k16 in range(4):
                    rows_v[fs, j, pl.ds(k16 * _L, _L)] = (
                        rows_v[fs, 0, pl.ds(k16 * _L, _L)])
            return 0

        def tail_do(out):
            lax.fori_loop(0, _HB, tbody, 0)
            flush(fs)
            return out + 1

        out = lax.cond(nb > 0, tail_do, lambda o: o, out)

        def dbody(i, _):
            wait_flush()
            return 0
        lax.fori_loop(0, out, dbody, 0)

    return k


def kernel(x, table):
    B0, H, W, C = x.shape
    V, D = table.shape
    flat = x.astype(jnp.int32).reshape(-1)
    B = flat.shape[0]
    tableT = table.T
    out128 = _make_lookup(V, D, B)(tableT, flat)
    return out128[:, :D].reshape(B0, H, W, D)
